# baseline (speedup 1.0000x reference)
import functools

import jax
import jax.numpy as jnp
from jax import lax
from jax.experimental import pallas as pl
from jax.experimental.pallas import tpu as pltpu
from jax.experimental.pallas import tpu_sc as plsc

N = 10000
E = 320000
D = 128
T = 4
R = 16
NR = N * R

NC = 2
NS = 16
NW = NC * NS
EB = E // NW
CK = 80
NCH = EB // CK
ROWS_PER_TILE = N // NS
CNT_PER_TILE = NR // NS

_mesh = plsc.VectorSubcoreMesh(core_axis_name="c", subcore_axis_name="s")


def _zero_vmem_1d(ref, nwords):
    def body(i, _):
        ref[pl.ds(i * 16, 16)] = jnp.zeros((16,), jnp.float32)
        return 0
    lax.fori_loop(0, nwords // 16, body, 0)


@functools.partial(
    pl.kernel,
    out_type=(
        jax.ShapeDtypeStruct((NW, NCH, CK), jnp.int32),
        jax.ShapeDtypeStruct((NC * NR,), jnp.float32),
    ),
    mesh=_mesh,
    compiler_params=pltpu.CompilerParams(needs_layout_passes=False),
    scratch_types=[
        pltpu.VMEM((NCH, CK), jnp.int32),
        pltpu.VMEM((NCH, CK), jnp.int32),
        pltpu.VMEM((NCH, CK), jnp.int32),
        pltpu.VMEM((N,), jnp.int32),
        pltpu.VMEM((CK,), jnp.float32),
        pltpu.VMEM((2000,), jnp.float32),
        pltpu.VMEM_SHARED((NR,), jnp.float32),
        pltpu.VMEM_SHARED((N,), jnp.int32),
        pltpu.SemaphoreType.DMA,
    ],
)
def _prep_kernel(src_hbm, tgt_hbm, nt_hbm, seg_hbm, cnt_hbm,
                 src_v, tgt_v, seg_v, nt_v, ones_v, z_v, cnt_sp, nt_sp, sem):
    c = lax.axis_index("c")
    s = lax.axis_index("s")
    wid = c * NS + s

    @pl.when(s == 0)
    def _():
        pltpu.sync_copy(nt_hbm, nt_v)
        pltpu.sync_copy(nt_v, nt_sp)
    _zero_vmem_1d(z_v, 2000)
    for q in range(5):
        pltpu.sync_copy(z_v, cnt_sp.at[pl.ds(s * CNT_PER_TILE + q * 2000, 2000)])

    for q in range(CK // 16):
        ones_v[pl.ds(q * 16, 16)] = jnp.ones((16,), jnp.float32)

    plsc.subcore_barrier()
    @pl.when(s != 0)
    def _():
        pltpu.sync_copy(nt_sp, nt_v)
    pltpu.sync_copy(src_hbm.at[wid], src_v)
    pltpu.sync_copy(tgt_hbm.at[wid], tgt_v)

    with jax.named_scope("seg_compute"):
        def comp_chunk(i, _):
            for b in range(CK // 16):
                sl = pl.ds(b * 16, 16)
                ids = src_v[i, sl]
                idt = tgt_v[i, sl]
                nts = plsc.load_gather(nt_v, [ids])
                ntt = plsc.load_gather(nt_v, [idt])
                seg_v[i, sl] = idt * R + ntt * T + nts
            return 0
        lax.fori_loop(0, NCH, comp_chunk, 0)

    with jax.named_scope("hist_fire"):
        def scat_chunk(i, _):
            pltpu.async_copy(ones_v, cnt_sp.at[seg_v.at[i]], sem, add=True)
            return 0
        lax.fori_loop(0, NCH, scat_chunk, 0)
    with jax.named_scope("hist_drain"):
        def scat_drain(i, _):
            pltpu.make_async_copy(ones_v, cnt_sp.at[seg_v.at[0]], sem).wait()
            return 0
        lax.fori_loop(0, NCH, scat_drain, 0)

    with jax.named_scope("seg_writeout"):
        pltpu.sync_copy(seg_v, seg_hbm.at[wid])
    plsc.subcore_barrier()
    def cout(q, _):
        o = s * CNT_PER_TILE + q * 2000
        pltpu.sync_copy(cnt_sp.at[pl.ds(o, 2000)], z_v)
        pltpu.sync_copy(z_v, cnt_hbm.at[pl.ds(c * NR + o, 2000)])
        return 0
    lax.fori_loop(0, CNT_PER_TILE // 2000, cout, 0)


@functools.partial(
    pl.kernel,
    out_type=(
        jax.ShapeDtypeStruct((NW, NCH, CK), jnp.float32),
        jax.ShapeDtypeStruct((NW, NCH, CK), jnp.float32),
    ),
    mesh=_mesh,
    compiler_params=pltpu.CompilerParams(needs_layout_passes=False),
    scratch_types=[
        pltpu.VMEM((NCH, CK), jnp.int32),
        pltpu.VMEM((NCH, CK), jnp.float32),
        pltpu.VMEM((NCH, CK), jnp.float32),
        pltpu.VMEM((CK,), jnp.float32),
        pltpu.VMEM((CK,), jnp.float32),
        pltpu.VMEM((R,), jnp.float32),
        pltpu.VMEM((R,), jnp.float32),
        pltpu.SemaphoreType.DMA,
        pltpu.SemaphoreType.DMA,
    ],
)
def _weight_kernel(seg_hbm, cnt0_hbm, cnt1_hbm, comp1_hbm, comp2_hbm,
                   w1_hbm, w2_hbm,
                   seg_v, w1_v, w2_v, p0_v, p1_v, c1_v, c2_v, sem0, sem1):
    c = lax.axis_index("c")
    s = lax.axis_index("s")
    wid = c * NS + s

    pltpu.sync_copy(seg_hbm.at[wid], seg_v)
    pltpu.sync_copy(comp1_hbm, c1_v)
    pltpu.sync_copy(comp2_hbm, c2_v)

    def chunk(i, _):
        d0 = pltpu.async_copy(cnt0_hbm.at[seg_v.at[i]], p0_v, sem0)
        d1 = pltpu.async_copy(cnt1_hbm.at[seg_v.at[i]], p1_v, sem1)
        d0.wait()
        d1.wait()
        for b in range(CK // 16):
            sl = pl.ds(b * 16, 16)
            cnt = p0_v[sl] + p1_v[sl]
            inv = 1.0 / jnp.maximum(cnt, 1.0)
            et = jnp.bitwise_and(seg_v[i, sl], R - 1)
            w1_v[i, sl] = plsc.load_gather(c1_v, [et]) * inv
            w2_v[i, sl] = plsc.load_gather(c2_v, [et]) * inv
        return 0
    lax.fori_loop(0, NCH, chunk, 0)

    pltpu.sync_copy(w1_v, w1_hbm.at[wid])
    pltpu.sync_copy(w2_v, w2_hbm.at[wid])


DH = D // 2
NCH2 = E // NS // CK
NB = 5


@functools.partial(
    pl.kernel,
    out_type=jax.ShapeDtypeStruct((NC, N, DH), jnp.float32),
    mesh=_mesh,
    compiler_params=pltpu.CompilerParams(needs_layout_passes=False,
                                         use_tc_tiling_on_sc=False),
    scratch_types=[
        pltpu.VMEM((NCH2, CK), jnp.int32),
        pltpu.VMEM((NCH2, CK), jnp.int32),
        pltpu.VMEM((NCH2, CK), jnp.float32),
        [pltpu.VMEM((CK, DH), jnp.float32)] * NB,
        pltpu.VMEM((16, DH), jnp.float32),
        pltpu.VMEM_SHARED((N, DH), jnp.float32),
        [pltpu.SemaphoreType.DMA] * NB,
        [pltpu.SemaphoreType.DMA] * NB,
    ],
)
def _agg_kernel(xab_hbm, src_hbm, tgt_hbm, w_hbm, out_hbm,
                src_v, tgt_v, w_v, rows, z_v, acc_sp, sem_g, sem_s):
    c = lax.axis_index("c")
    s = lax.axis_index("s")
    x_h = xab_hbm.at[c]

    def zfill(i, _):
        for d in range(DH // 16):
            z_v[i, pl.ds(d * 16, 16)] = jnp.zeros((16,), jnp.float32)
        return 0
    lax.fori_loop(0, 16, zfill, 0)
    def zrows(q, _):
        pltpu.sync_copy(z_v, acc_sp.at[pl.ds(s * 624 + q * 16, 16)])
        return 0
    lax.fori_loop(0, 624 // 16, zrows, 0)
    @pl.when(s == NS - 1)
    def _():
        pltpu.sync_copy(z_v, acc_sp.at[pl.ds(9984, 16)])

    pltpu.sync_copy(src_hbm.at[s], src_v)
    pltpu.sync_copy(tgt_hbm.at[s], tgt_v)
    pltpu.sync_copy(w_hbm.at[s], w_v)

    plsc.subcore_barrier()

    def scale(buf, i):
        for g in range(CK // 16):
            wv16 = w_v[i, pl.ds(g * 16, 16)]
            for j in range(16):
                wj = jnp.full((16,), wv16[j])
                row = g * 16 + j
                for d in range(DH // 16):
                    sl = pl.ds(d * 16, 16)
                    buf[row, sl] = buf[row, sl] * wj

    for b in range(3):
        pltpu.async_copy(x_h.at[src_v.at[b]], rows[b], sem_g[b])

    def group(q, _):
        for k in range(NB):
            i = NB * q + k
            pltpu.make_async_copy(x_h.at[src_v.at[0]], rows[k],
                                  sem_g[k]).wait()
            scale(rows[k], i)
            pltpu.async_copy(rows[k], acc_sp.at[tgt_v.at[i]], sem_s[k],
                             add=True)
            k3 = (k + 3) % NB
            @pl.when(i >= 2)
            def _():
                pltpu.make_async_copy(rows[k3], acc_sp.at[tgt_v.at[0]],
                                      sem_s[k3]).wait()
            @pl.when(i + 3 < NCH2)
            def _():
                pltpu.async_copy(x_h.at[src_v.at[i + 3]], rows[k3],
                                 sem_g[k3])
        return 0
    lax.fori_loop(0, NCH2 // NB, group, 0)
    pltpu.make_async_copy(rows[3], acc_sp.at[tgt_v.at[0]], sem_s[3]).wait()
    pltpu.make_async_copy(rows[4], acc_sp.at[tgt_v.at[0]], sem_s[4]).wait()

    plsc.subcore_barrier()
    def wout(q, _):
        r0 = s * 624 + q * CK
        pltpu.sync_copy(acc_sp.at[pl.ds(r0, CK)], rows[0])
        pltpu.sync_copy(rows[0], out_hbm.at[c, pl.ds(r0, CK)])
        return 0
    lax.fori_loop(0, 7, wout, 0)
    r0 = s * 624 + 560
    pltpu.sync_copy(acc_sp.at[pl.ds(r0, 64)], rows[1].at[pl.ds(0, 64)])
    pltpu.sync_copy(rows[1].at[pl.ds(0, 64)], out_hbm.at[c, pl.ds(r0, 64)])
    @pl.when(s == NS - 1)
    def _():
        pltpu.sync_copy(acc_sp.at[pl.ds(9984, 16)], rows[2].at[pl.ds(0, 16)])
        pltpu.sync_copy(rows[2].at[pl.ds(0, 16)],
                        out_hbm.at[c, pl.ds(9984, 16)])


_BM = 400


def _dense_body(p_ref, x_ref, va_ref, vb_ref, root_ref, b_ref, o_ref):
    za = jnp.dot(p_ref[0], va_ref[...], preferred_element_type=jnp.float32)
    zb = jnp.dot(p_ref[1], vb_ref[...], preferred_element_type=jnp.float32)
    zr = jnp.dot(x_ref[...], root_ref[...], preferred_element_type=jnp.float32)
    z = za + zb + zr + b_ref[...]
    o_ref[...] = 1.0 / (1.0 + jnp.exp(-z))


def _dense(pab, x, v, root, bias):
    return pl.pallas_call(
        _dense_body,
        grid=(N // _BM,),
        in_specs=[
            pl.BlockSpec((NC, _BM, DH), lambda i: (0, i, 0)),
            pl.BlockSpec((_BM, D), lambda i: (i, 0)),
            pl.BlockSpec((DH, D), lambda i: (0, 0)),
            pl.BlockSpec((DH, D), lambda i: (0, 0)),
            pl.BlockSpec((D, D), lambda i: (0, 0)),
            pl.BlockSpec((1, D), lambda i: (0, 0)),
        ],
        out_specs=pl.BlockSpec((_BM, D), lambda i: (i, 0)),
        out_shape=jax.ShapeDtypeStruct((N, D), jnp.float32),
    )(pab, x, v[:DH], v[DH:], root, bias.reshape(1, D))


def kernel(x, edge_index, node_type, V1, comp1, root1, bias1,
           V2, comp2, root2, bias2):
    src = edge_index[0].astype(jnp.int32)
    tgt = edge_index[1].astype(jnp.int32)
    src32 = src.reshape(NW, NCH, CK)
    tgt32 = tgt.reshape(NW, NCH, CK)
    src16 = src.reshape(NS, NCH2, CK)
    tgt16 = tgt.reshape(NS, NCH2, CK)
    nt = node_type.astype(jnp.int32)

    seg, cnt = _prep_kernel(src32, tgt32, nt)
    w1, w2 = _weight_kernel(seg, cnt[:NR], cnt[NR:],
                            comp1.reshape(R), comp2.reshape(R))

    xab1 = jnp.stack([x[:, :DH], x[:, DH:]])
    pab1 = _agg_kernel(xab1, src16, tgt16, w1.reshape(NS, NCH2, CK))
    x1 = _dense(pab1, x, V1[0], root1, bias1)
    xab2 = jnp.stack([x1[:, :DH], x1[:, DH:]])
    pab2 = _agg_kernel(xab2, src16, tgt16, w2.reshape(NS, NCH2, CK))
    x2 = _dense(pab2, x1, V2[0], root2, bias2)
    return jnp.concatenate([x1, x2], axis=1)


# speedup vs baseline: 1.1646x; 1.1646x over previous
import functools

import jax
import jax.numpy as jnp
from jax import lax
from jax.experimental import pallas as pl
from jax.experimental.pallas import tpu as pltpu
from jax.experimental.pallas import tpu_sc as plsc

N = 10000
E = 320000
D = 128
T = 4
R = 16
NR = N * R

NC = 2
NS = 16
CK = 80
EB = E // NS
NCH = EB // CK
CNT_PER_TILE = NR // NS
DH = D // 2
NB = 5

_mesh = plsc.VectorSubcoreMesh(core_axis_name="c", subcore_axis_name="s")
_params = pltpu.CompilerParams(needs_layout_passes=False,
                               use_tc_tiling_on_sc=False)


def _fill_zeros(z_v):
    def zfill(i, _):
        for d in range(DH // 16):
            z_v[i, pl.ds(d * 16, 16)] = jnp.zeros((16,), jnp.float32)
        return 0
    lax.fori_loop(0, 16, zfill, 0)


def _zero_acc(z_v, acc_sp, s):
    def zrows(q, _):
        pltpu.sync_copy(z_v, acc_sp.at[pl.ds(s * 624 + q * 16, 16)])
        return 0
    lax.fori_loop(0, 624 // 16, zrows, 0)
    @pl.when(s == NS - 1)
    def _():
        pltpu.sync_copy(z_v, acc_sp.at[pl.ds(9984, 16)])


def _agg_loop(x_h, src_v, tgt_v, rows, acc_sp, sem_g, sem_s, weight16):
    def scale(buf, i):
        for g in range(CK // 16):
            w16 = weight16(i, g)
            for j in range(16):
                wj = jnp.full((16,), w16[j])
                row = g * 16 + j
                for d in range(DH // 16):
                    sl = pl.ds(d * 16, 16)
                    buf[row, sl] = buf[row, sl] * wj

    for b in range(3):
        pltpu.async_copy(x_h.at[src_v.at[pl.ds(b * CK, CK)]], rows[b],
                         sem_g[b])

    def group(q, _):
        for k in range(NB):
            i = NB * q + k
            pltpu.make_async_copy(x_h.at[src_v.at[pl.ds(0, CK)]], rows[k],
                                  sem_g[k]).wait()
            scale(rows[k], i)
            pltpu.async_copy(rows[k], acc_sp.at[tgt_v.at[i]], sem_s[k],
                             add=True)
            k3 = (k + 3) % NB
            @pl.when(i >= 2)
            def _():
                pltpu.make_async_copy(rows[k3], acc_sp.at[tgt_v.at[0]],
                                      sem_s[k3]).wait()
            @pl.when(i + 3 < NCH)
            def _():
                pltpu.async_copy(x_h.at[src_v.at[pl.ds((i + 3) * CK, CK)]],
                                 rows[k3], sem_g[k3])
        return 0
    lax.fori_loop(0, NCH // NB, group, 0)
    pltpu.make_async_copy(rows[3], acc_sp.at[tgt_v.at[0]], sem_s[3]).wait()
    pltpu.make_async_copy(rows[4], acc_sp.at[tgt_v.at[0]], sem_s[4]).wait()


def _writeout(acc_sp, out_hbm, rows, c, s):
    def wout(q, _):
        r0 = s * 624 + q * CK
        pltpu.sync_copy(acc_sp.at[pl.ds(r0, CK)], rows[0])
        pltpu.sync_copy(rows[0], out_hbm.at[c, pl.ds(r0, CK)])
        return 0
    lax.fori_loop(0, 7, wout, 0)
    r0 = s * 624 + 560
    pltpu.sync_copy(acc_sp.at[pl.ds(r0, 64)], rows[1].at[pl.ds(0, 64)])
    pltpu.sync_copy(rows[1].at[pl.ds(0, 64)], out_hbm.at[c, pl.ds(r0, 64)])
    @pl.when(s == NS - 1)
    def _():
        pltpu.sync_copy(acc_sp.at[pl.ds(9984, 16)], rows[2].at[pl.ds(0, 16)])
        pltpu.sync_copy(rows[2].at[pl.ds(0, 16)],
                        out_hbm.at[c, pl.ds(9984, 16)])


@functools.partial(
    pl.kernel,
    out_type=(
        jax.ShapeDtypeStruct((NS, EB), jnp.float32),
        jax.ShapeDtypeStruct((NS, EB), jnp.float32),
    ),
    mesh=_mesh,
    compiler_params=_params,
    scratch_types=[
        pltpu.VMEM((EB,), jnp.int32),
        pltpu.VMEM((EB,), jnp.int32),
        pltpu.VMEM((NCH, CK), jnp.int32),
        pltpu.VMEM((EB,), jnp.float32),
        pltpu.VMEM((EB,), jnp.float32),
        pltpu.VMEM((2000,), jnp.float32),
        pltpu.VMEM((N,), jnp.int32),
        pltpu.VMEM((CK,), jnp.float32),
        pltpu.VMEM((CK,), jnp.float32),
        pltpu.VMEM((CK,), jnp.float32),
        pltpu.VMEM((R,), jnp.float32),
        pltpu.VMEM((R,), jnp.float32),
        pltpu.VMEM_SHARED((NR,), jnp.float32),
        pltpu.VMEM_SHARED((N,), jnp.int32),
        pltpu.SemaphoreType.DMA,
        pltpu.SemaphoreType.DMA,
        pltpu.SemaphoreType.DMA,
    ],
)
def _prep_kernel(src_hbm, tgt_hbm, nt_hbm, comp1_hbm, comp2_hbm,
                 w1_hbm, w2_hbm,
                 src_v, tgt_v, seg_v, w1_v, w2_v, zf_v, nt_v, ones_v,
                 cn0_v, cn1_v, c1_v, c2_v, cnt_sp, nt_sp,
                 sem_h, sem_c0, sem_c1):
    c = lax.axis_index("c")
    s = lax.axis_index("s")

    @pl.when(s == 0)
    def _():
        pltpu.sync_copy(nt_hbm, nt_v)
        pltpu.sync_copy(nt_v, nt_sp)
    def zf(i, _):
        zf_v[pl.ds(i * 16, 16)] = jnp.zeros((16,), jnp.float32)
        return 0
    lax.fori_loop(0, 125, zf, 0)
    for q in range(CNT_PER_TILE // 2000):
        pltpu.sync_copy(zf_v,
                        cnt_sp.at[pl.ds(s * CNT_PER_TILE + q * 2000, 2000)])
    for q in range(CK // 16):
        ones_v[pl.ds(q * 16, 16)] = jnp.ones((16,), jnp.float32)
    plsc.subcore_barrier()

    @pl.when(s != 0)
    def _():
        pltpu.sync_copy(nt_sp, nt_v)
    pltpu.sync_copy(src_hbm.at[s], src_v)
    pltpu.sync_copy(tgt_hbm.at[s], tgt_v)
    pltpu.sync_copy(comp1_hbm, c1_v)
    pltpu.sync_copy(comp2_hbm, c2_v)

    def comp_chunk(i, _):
        for b in range(CK // 16):
            sl = pl.ds(i * CK + b * 16, 16)
            ids = src_v[sl]
            idt = tgt_v[sl]
            nts = plsc.load_gather(nt_v, [ids])
            ntt = plsc.load_gather(nt_v, [idt])
            seg_v[i, pl.ds(b * 16, 16)] = idt * R + ntt * T + nts
        return 0
    lax.fori_loop(0, NCH, comp_chunk, 0)

    def scat_chunk(i, _):
        pltpu.async_copy(ones_v, cnt_sp.at[seg_v.at[i]], sem_h, add=True)
        return 0
    lax.fori_loop(0, NCH, scat_chunk, 0)
    def scat_drain(i, _):
        pltpu.make_async_copy(ones_v, cnt_sp.at[seg_v.at[0]], sem_h).wait()
        return 0
    lax.fori_loop(0, NCH, scat_drain, 0)
    plsc.subcore_barrier()

    pltpu.async_copy(cnt_sp.at[seg_v.at[0]], cn0_v, sem_c0)
    pltpu.async_copy(cnt_sp.at[seg_v.at[1]], cn1_v, sem_c1)
    def weights(i, cn):
        for b in range(CK // 16):
            sl = pl.ds(b * 16, 16)
            inv = 1.0 / jnp.maximum(cn[sl], 1.0)
            et = jnp.bitwise_and(seg_v[i, sl], R - 1)
            fl = pl.ds(i * CK + b * 16, 16)
            w1_v[fl] = plsc.load_gather(c1_v, [et]) * inv
            w2_v[fl] = plsc.load_gather(c2_v, [et]) * inv
    def wpair(p, _):
        i0 = 2 * p
        i1 = 2 * p + 1
        pltpu.make_async_copy(cnt_sp.at[seg_v.at[0]], cn0_v, sem_c0).wait()
        weights(i0, cn0_v)
        @pl.when(i0 + 2 < NCH)
        def _():
            pltpu.async_copy(cnt_sp.at[seg_v.at[i0 + 2]], cn0_v, sem_c0)
        pltpu.make_async_copy(cnt_sp.at[seg_v.at[0]], cn1_v, sem_c1).wait()
        weights(i1, cn1_v)
        @pl.when(i1 + 2 < NCH)
        def _():
            pltpu.async_copy(cnt_sp.at[seg_v.at[i1 + 2]], cn1_v, sem_c1)
        return 0
    lax.fori_loop(0, NCH // 2, wpair, 0)
    @pl.when(c == 0)
    def _():
        pltpu.sync_copy(w1_v, w1_hbm.at[s])
        pltpu.sync_copy(w2_v, w2_hbm.at[s])


@functools.partial(
    pl.kernel,
    out_type=jax.ShapeDtypeStruct((NC, N, DH), jnp.float32),
    mesh=_mesh,
    compiler_params=_params,
    scratch_types=[
        pltpu.VMEM((EB,), jnp.int32),
        pltpu.VMEM((NCH, CK), jnp.int32),
        pltpu.VMEM((EB,), jnp.float32),
        [pltpu.VMEM((CK, DH), jnp.float32)] * NB,
        pltpu.VMEM((16, DH), jnp.float32),
        pltpu.VMEM_SHARED((N, DH), jnp.float32),
        [pltpu.SemaphoreType.DMA] * NB,
        [pltpu.SemaphoreType.DMA] * NB,
    ],
)
def _aggc_kernel(xab_hbm, srcf_hbm, tgt_hbm, w_hbm, pab_hbm,
                 src_v, tgt_v, w_v, rows, z_v, acc_sp, sem_g, sem_s):
    c = lax.axis_index("c")
    s = lax.axis_index("s")
    x_h = xab_hbm.at[c]

    _fill_zeros(z_v)
    _zero_acc(z_v, acc_sp, s)
    pltpu.sync_copy(srcf_hbm.at[s], src_v)
    pltpu.sync_copy(tgt_hbm.at[s], tgt_v)
    pltpu.sync_copy(w_hbm.at[s], w_v)
    plsc.subcore_barrier()

    def weight16(i, g):
        return w_v[pl.ds(i * CK + g * 16, 16)]
    _agg_loop(x_h, src_v, tgt_v, rows, acc_sp, sem_g, sem_s, weight16)

    plsc.subcore_barrier()
    _writeout(acc_sp, pab_hbm, rows, c, s)


_BM = 400


def _dense_body(p_ref, x_ref, va_ref, vb_ref, root_ref, b_ref, o_ref):
    za = jnp.dot(p_ref[0], va_ref[...], preferred_element_type=jnp.float32)
    zb = jnp.dot(p_ref[1], vb_ref[...], preferred_element_type=jnp.float32)
    zr = jnp.dot(x_ref[...], root_ref[...], preferred_element_type=jnp.float32)
    z = za + zb + zr + b_ref[...]
    o_ref[...] = 1.0 / (1.0 + jnp.exp(-z))


def _dense(pab, x, v, root, bias):
    return pl.pallas_call(
        _dense_body,
        grid=(N // _BM,),
        in_specs=[
            pl.BlockSpec((NC, _BM, DH), lambda i: (0, i, 0)),
            pl.BlockSpec((_BM, D), lambda i: (i, 0)),
            pl.BlockSpec((DH, D), lambda i: (0, 0)),
            pl.BlockSpec((DH, D), lambda i: (0, 0)),
            pl.BlockSpec((D, D), lambda i: (0, 0)),
            pl.BlockSpec((1, D), lambda i: (0, 0)),
        ],
        out_specs=pl.BlockSpec((_BM, D), lambda i: (i, 0)),
        out_shape=jax.ShapeDtypeStruct((N, D), jnp.float32),
    )(pab, x, v[:DH], v[DH:], root, bias.reshape(1, D))


def kernel(x, edge_index, node_type, V1, comp1, root1, bias1,
           V2, comp2, root2, bias2):
    srcf = edge_index[0].astype(jnp.int32).reshape(NS, EB)
    tgtf = edge_index[1].astype(jnp.int32).reshape(NS, EB)
    tgt16 = tgtf.reshape(NS, NCH, CK)
    nt = node_type.astype(jnp.int32)

    w1, w2 = _prep_kernel(srcf, tgtf, nt, comp1.reshape(R), comp2.reshape(R))
    xab1 = jnp.stack([x[:, :DH], x[:, DH:]])
    pab1 = _aggc_kernel(xab1, srcf, tgt16, w1)
    x1 = _dense(pab1, x, V1[0], root1, bias1)
    xab2 = jnp.stack([x1[:, :DH], x1[:, DH:]])
    pab2 = _aggc_kernel(xab2, srcf, tgt16, w2)
    x2 = _dense(pab2, x1, V2[0], root2, bias2)
    return jnp.concatenate([x1, x2], axis=1)


# speedup vs baseline: 1.2330x; 1.0587x over previous
import functools

import jax
import jax.numpy as jnp
from jax import lax
from jax.experimental import pallas as pl
from jax.experimental.pallas import tpu as pltpu
from jax.experimental.pallas import tpu_sc as plsc

N = 10000
E = 320000
D = 128
T = 4
R = 16
NR = N * R

NC = 2
NS = 16
CK = 80
EB = E // NS
NCH = EB // CK
CNT_PER_TILE = NR // NS
DH = D // 2
NB = 5

_mesh = plsc.VectorSubcoreMesh(core_axis_name="c", subcore_axis_name="s")
_params = pltpu.CompilerParams(needs_layout_passes=False,
                               use_tc_tiling_on_sc=False)


def _fill_zeros(z_v):
    def zfill(i, _):
        for d in range(DH // 16):
            z_v[i, pl.ds(d * 16, 16)] = jnp.zeros((16,), jnp.float32)
        return 0
    lax.fori_loop(0, 16, zfill, 0)


def _zero_acc(z_v, acc_sp, s, sem):
    def zrows(q, _):
        pltpu.async_copy(z_v, acc_sp.at[pl.ds(s * 624 + q * 16, 16)], sem)
        return 0
    lax.fori_loop(0, 624 // 16, zrows, 0)
    @pl.when(s == NS - 1)
    def _():
        pltpu.async_copy(z_v, acc_sp.at[pl.ds(9984, 16)], sem)
    def zdrain(q, _):
        pltpu.make_async_copy(z_v, acc_sp.at[pl.ds(0, 16)], sem).wait()
        return 0
    lax.fori_loop(0, 624 // 16, zdrain, 0)
    @pl.when(s == NS - 1)
    def _():
        pltpu.make_async_copy(z_v, acc_sp.at[pl.ds(0, 16)], sem).wait()


def _agg_loop(x_h, src_v, tgt_v, rows, acc_sp, sem_g, sem_s, weight16):
    def scale(buf, i):
        for g in range(CK // 16):
            w16 = weight16(i, g)
            for j in range(16):
                wj = jnp.full((16,), w16[j])
                row = g * 16 + j
                for d in range(DH // 16):
                    sl = pl.ds(d * 16, 16)
                    buf[row, sl] = buf[row, sl] * wj

    for b in range(3):
        pltpu.async_copy(x_h.at[src_v.at[pl.ds(b * CK, CK)]], rows[b],
                         sem_g[b])

    def group(q, _):
        for k in range(NB):
            i = NB * q + k
            pltpu.make_async_copy(x_h.at[src_v.at[pl.ds(0, CK)]], rows[k],
                                  sem_g[k]).wait()
            scale(rows[k], i)
            pltpu.async_copy(rows[k], acc_sp.at[tgt_v.at[i]], sem_s[k],
                             add=True)
            k3 = (k + 3) % NB
            @pl.when(i >= 2)
            def _():
                pltpu.make_async_copy(rows[k3], acc_sp.at[tgt_v.at[0]],
                                      sem_s[k3]).wait()
            @pl.when(i + 3 < NCH)
            def _():
                pltpu.async_copy(x_h.at[src_v.at[pl.ds((i + 3) * CK, CK)]],
                                 rows[k3], sem_g[k3])
        return 0
    lax.fori_loop(0, NCH // NB, group, 0)
    pltpu.make_async_copy(rows[3], acc_sp.at[tgt_v.at[0]], sem_s[3]).wait()
    pltpu.make_async_copy(rows[4], acc_sp.at[tgt_v.at[0]], sem_s[4]).wait()


def _writeout(acc_sp, out_hbm, rows, c, s, sem_g, sem_s):
    def r0(q):
        return s * 624 + q * CK
    for q in range(5):
        pltpu.async_copy(acc_sp.at[pl.ds(r0(q), CK)], rows[q], sem_g[q])
    for q in range(7):
        b = q % 5
        pltpu.make_async_copy(acc_sp.at[pl.ds(0, CK)], rows[b],
                              sem_g[b]).wait()
        pltpu.async_copy(rows[b], out_hbm.at[c, pl.ds(r0(q), CK)], sem_s[b])
        if q + 5 < 7:
            pltpu.make_async_copy(rows[b], out_hbm.at[c, pl.ds(0, CK)],
                                  sem_s[b]).wait()
            pltpu.async_copy(acc_sp.at[pl.ds(r0(q + 5), CK)], rows[b],
                             sem_g[b])
    for q in range(2, 7):
        pltpu.make_async_copy(rows[q % 5], out_hbm.at[c, pl.ds(0, CK)],
                              sem_s[q % 5]).wait()
    rt = s * 624 + 560
    pltpu.sync_copy(acc_sp.at[pl.ds(rt, 64)], rows[2].at[pl.ds(0, 64)])
    pltpu.sync_copy(rows[2].at[pl.ds(0, 64)], out_hbm.at[c, pl.ds(rt, 64)])
    @pl.when(s == NS - 1)
    def _():
        pltpu.sync_copy(acc_sp.at[pl.ds(9984, 16)], rows[3].at[pl.ds(0, 16)])
        pltpu.sync_copy(rows[3].at[pl.ds(0, 16)],
                        out_hbm.at[c, pl.ds(9984, 16)])


@functools.partial(
    pl.kernel,
    out_type=(
        jax.ShapeDtypeStruct((NS, EB), jnp.float32),
        jax.ShapeDtypeStruct((NS, EB), jnp.float32),
    ),
    mesh=_mesh,
    compiler_params=_params,
    scratch_types=[
        pltpu.VMEM((EB,), jnp.int32),
        pltpu.VMEM((EB,), jnp.int32),
        pltpu.VMEM((NCH, CK), jnp.int32),
        pltpu.VMEM((EB,), jnp.float32),
        pltpu.VMEM((EB,), jnp.float32),
        pltpu.VMEM((2000,), jnp.float32),
        pltpu.VMEM((N,), jnp.int32),
        pltpu.VMEM((CK,), jnp.float32),
        pltpu.VMEM((CK,), jnp.float32),
        pltpu.VMEM((CK,), jnp.float32),
        pltpu.VMEM((R,), jnp.float32),
        pltpu.VMEM((R,), jnp.float32),
        pltpu.VMEM_SHARED((NR,), jnp.float32),
        pltpu.VMEM_SHARED((N,), jnp.int32),
        pltpu.SemaphoreType.DMA,
        pltpu.SemaphoreType.DMA,
        pltpu.SemaphoreType.DMA,
    ],
)
def _prep_kernel(src_hbm, tgt_hbm, nt_hbm, comp1_hbm, comp2_hbm,
                 w1_hbm, w2_hbm,
                 src_v, tgt_v, seg_v, w1_v, w2_v, zf_v, nt_v, ones_v,
                 cn0_v, cn1_v, c1_v, c2_v, cnt_sp, nt_sp,
                 sem_h, sem_c0, sem_c1):
    c = lax.axis_index("c")
    s = lax.axis_index("s")

    @pl.when(s == 0)
    def _():
        pltpu.sync_copy(nt_hbm, nt_v)
        pltpu.sync_copy(nt_v, nt_sp)
    def zf(i, _):
        zf_v[pl.ds(i * 16, 16)] = jnp.zeros((16,), jnp.float32)
        return 0
    lax.fori_loop(0, 125, zf, 0)
    for q in range(CNT_PER_TILE // 2000):
        pltpu.sync_copy(zf_v,
                        cnt_sp.at[pl.ds(s * CNT_PER_TILE + q * 2000, 2000)])
    for q in range(CK // 16):
        ones_v[pl.ds(q * 16, 16)] = jnp.ones((16,), jnp.float32)
    plsc.subcore_barrier()

    @pl.when(s != 0)
    def _():
        pltpu.sync_copy(nt_sp, nt_v)
    pltpu.sync_copy(src_hbm.at[s], src_v)
    pltpu.sync_copy(tgt_hbm.at[s], tgt_v)
    pltpu.sync_copy(comp1_hbm, c1_v)
    pltpu.sync_copy(comp2_hbm, c2_v)

    def comp_chunk(i, _):
        for b in range(CK // 16):
            sl = pl.ds(i * CK + b * 16, 16)
            ids = src_v[sl]
            idt = tgt_v[sl]
            nts = plsc.load_gather(nt_v, [ids])
            ntt = plsc.load_gather(nt_v, [idt])
            seg_v[i, pl.ds(b * 16, 16)] = idt * R + ntt * T + nts
        return 0
    lax.fori_loop(0, NCH, comp_chunk, 0)

    def scat_chunk(i, _):
        pltpu.async_copy(ones_v, cnt_sp.at[seg_v.at[i]], sem_h, add=True)
        return 0
    lax.fori_loop(0, NCH, scat_chunk, 0)
    def scat_drain(i, _):
        pltpu.make_async_copy(ones_v, cnt_sp.at[seg_v.at[0]], sem_h).wait()
        return 0
    lax.fori_loop(0, NCH, scat_drain, 0)
    plsc.subcore_barrier()

    pltpu.async_copy(cnt_sp.at[seg_v.at[0]], cn0_v, sem_c0)
    pltpu.async_copy(cnt_sp.at[seg_v.at[1]], cn1_v, sem_c1)
    def weights(i, cn):
        for b in range(CK // 16):
            sl = pl.ds(b * 16, 16)
            inv = 1.0 / jnp.maximum(cn[sl], 1.0)
            et = jnp.bitwise_and(seg_v[i, sl], R - 1)
            fl = pl.ds(i * CK + b * 16, 16)
            w1_v[fl] = plsc.load_gather(c1_v, [et]) * inv
            w2_v[fl] = plsc.load_gather(c2_v, [et]) * inv
    def wpair(p, _):
        i0 = 2 * p
        i1 = 2 * p + 1
        pltpu.make_async_copy(cnt_sp.at[seg_v.at[0]], cn0_v, sem_c0).wait()
        weights(i0, cn0_v)
        @pl.when(i0 + 2 < NCH)
        def _():
            pltpu.async_copy(cnt_sp.at[seg_v.at[i0 + 2]], cn0_v, sem_c0)
        pltpu.make_async_copy(cnt_sp.at[seg_v.at[0]], cn1_v, sem_c1).wait()
        weights(i1, cn1_v)
        @pl.when(i1 + 2 < NCH)
        def _():
            pltpu.async_copy(cnt_sp.at[seg_v.at[i1 + 2]], cn1_v, sem_c1)
        return 0
    lax.fori_loop(0, NCH // 2, wpair, 0)
    @pl.when(c == 0)
    def _():
        pltpu.sync_copy(w1_v, w1_hbm.at[s])
        pltpu.sync_copy(w2_v, w2_hbm.at[s])


@functools.partial(
    pl.kernel,
    out_type=jax.ShapeDtypeStruct((NC, N, DH), jnp.float32),
    mesh=_mesh,
    compiler_params=_params,
    scratch_types=[
        pltpu.VMEM((EB,), jnp.int32),
        pltpu.VMEM((NCH, CK), jnp.int32),
        pltpu.VMEM((EB,), jnp.float32),
        [pltpu.VMEM((CK, DH), jnp.float32)] * NB,
        pltpu.VMEM((16, DH), jnp.float32),
        pltpu.VMEM_SHARED((N, DH), jnp.float32),
        [pltpu.SemaphoreType.DMA] * NB,
        [pltpu.SemaphoreType.DMA] * NB,
    ],
)
def _aggc_kernel(xab_hbm, srcf_hbm, tgt_hbm, w_hbm, pab_hbm,
                 src_v, tgt_v, w_v, rows, z_v, acc_sp, sem_g, sem_s):
    c = lax.axis_index("c")
    s = lax.axis_index("s")
    x_h = xab_hbm.at[c]

    _fill_zeros(z_v)
    _zero_acc(z_v, acc_sp, s, sem_g[0])
    pltpu.sync_copy(srcf_hbm.at[s], src_v)
    pltpu.sync_copy(tgt_hbm.at[s], tgt_v)
    pltpu.sync_copy(w_hbm.at[s], w_v)
    plsc.subcore_barrier()

    def weight16(i, g):
        return w_v[pl.ds(i * CK + g * 16, 16)]
    _agg_loop(x_h, src_v, tgt_v, rows, acc_sp, sem_g, sem_s, weight16)

    plsc.subcore_barrier()
    _writeout(acc_sp, pab_hbm, rows, c, s, sem_g, sem_s)


_BM = 1000


def _dense_body(p_ref, x_ref, va_ref, vb_ref, root_ref, b_ref, o_ref):
    za = jnp.dot(p_ref[0], va_ref[...], preferred_element_type=jnp.float32)
    zb = jnp.dot(p_ref[1], vb_ref[...], preferred_element_type=jnp.float32)
    zr = jnp.dot(x_ref[...], root_ref[...], preferred_element_type=jnp.float32)
    z = za + zb + zr + b_ref[...]
    o_ref[...] = 1.0 / (1.0 + jnp.exp(-z))


def _dense(pab, x, v, root, bias):
    return pl.pallas_call(
        _dense_body,
        grid=(N // _BM,),
        in_specs=[
            pl.BlockSpec((NC, _BM, DH), lambda i: (0, i, 0)),
            pl.BlockSpec((_BM, D), lambda i: (i, 0)),
            pl.BlockSpec((DH, D), lambda i: (0, 0)),
            pl.BlockSpec((DH, D), lambda i: (0, 0)),
            pl.BlockSpec((D, D), lambda i: (0, 0)),
            pl.BlockSpec((1, D), lambda i: (0, 0)),
        ],
        out_specs=pl.BlockSpec((_BM, D), lambda i: (i, 0)),
        out_shape=jax.ShapeDtypeStruct((N, D), jnp.float32),
    )(pab, x, v[:DH], v[DH:], root, bias.reshape(1, D))


def kernel(x, edge_index, node_type, V1, comp1, root1, bias1,
           V2, comp2, root2, bias2):
    srcf = edge_index[0].astype(jnp.int32).reshape(NS, EB)
    tgtf = edge_index[1].astype(jnp.int32).reshape(NS, EB)
    tgt16 = tgtf.reshape(NS, NCH, CK)
    nt = node_type.astype(jnp.int32)

    w1, w2 = _prep_kernel(srcf, tgtf, nt, comp1.reshape(R), comp2.reshape(R))
    xab1 = jnp.stack([x[:, :DH], x[:, DH:]])
    pab1 = _aggc_kernel(xab1, srcf, tgt16, w1)
    x1 = _dense(pab1, x, V1[0], root1, bias1)
    xab2 = jnp.stack([x1[:, :DH], x1[:, DH:]])
    pab2 = _aggc_kernel(xab2, srcf, tgt16, w2)
    x2 = _dense(pab2, x1, V2[0], root2, bias2)
    return jnp.concatenate([x1, x2], axis=1)
